# Initial kernel scaffold; baseline (speedup 1.0000x reference)
import jax
import jax.numpy as jnp
from jax.experimental import pallas as pl


def kernel(tokens, graph, signs, W2, b2, g2, be2, W3, b3, g3, be3, hash_idx):
    raise NotImplementedError("write your pallas kernel here")


# speedup vs baseline: 2.8698x; 2.8698x over previous
import numpy as np
import jax
import jax.numpy as jnp
from jax import lax
from jax.experimental import pallas as pl
from jax.experimental.pallas import tpu as pltpu

EPS = 1e-5
B, N, D = 64, 196, 384
ESK = 768
DSEC = 512
SEC = D * (D + 1) // 2
KB = 7424
KSTEPS = 10
_BN_SCALE = 1.0 / np.sqrt(1.0 + EPS)
_INV_SQRT2 = 1.0 / np.sqrt(2.0)


def _gelu_exact(y):
    return 0.5 * y * (1.0 + lax.erf(y * _INV_SQRT2))


def _stage1_body(tokens_ref, graph_ref, m2n_ref, wc_ref):
    g = graph_ref[0]
    t = tokens_ref[0]
    deg = jnp.maximum(jnp.sum(g, axis=1, keepdims=True), EPS)
    dinv = lax.rsqrt(deg)
    gd = jnp.dot(g, dinv, preferred_element_type=jnp.float32)
    row = dinv * gd
    r_i = lax.broadcasted_iota(jnp.int32, (N, N), 0)
    c_i = lax.broadcasted_iota(jnp.int32, (N, N), 1)
    diag = jnp.sum(jnp.where(r_i == c_i, g, 0.0), axis=1, keepdims=True)
    trw = jnp.sum(diag * dinv * dinv)
    inv_trw = 1.0 / (trw + EPS)
    mu = lax.dot_general(row, t, (((0,), (0,)), ((), ())),
                         preferred_element_type=jnp.float32) * inv_trw
    tc = t - mu
    u = tc * dinv
    gu = jnp.dot(g, u, preferred_element_type=jnp.float32)
    m2 = lax.dot_general(u, gu, (((0,), (0,)), ((), ())),
                         preferred_element_type=jnp.float32)
    tr = jnp.sum(u * gu)
    z = m2 * (1.0 / (tr + EPS))
    rI = lax.broadcasted_iota(jnp.int32, (D, D), 0)
    cI = lax.broadcasted_iota(jnp.int32, (D, D), 1)
    eye = jnp.where(rI == cI, 1.0, 0.0)
    y = 1.5 * eye - 0.5 * z
    for _ in range(2):
        zy = jnp.dot(z, y, preferred_element_type=jnp.float32)
        y = 0.5 * jnp.dot(y, 3.0 * eye - zy,
                          preferred_element_type=jnp.float32)
    m2n_ref[0] = y * (1.0 / jnp.sqrt(tr + EPS))

    wc = lax.dot_general(row, tc, (((0,), (0,)), ((), ())),
                         preferred_element_type=jnp.float32) * inv_trw
    wc_ref[0] = wc


def _stage3_body(wc_ref, signs_ref, hash_ref, w3_ref, b3_ref, g3_ref,
                 be3_ref, h3_ref):
    wc = wc_ref[...]
    m_i = lax.broadcasted_iota(jnp.int32, (ESK, D), 0)
    res = jnp.ones((B, ESK), jnp.float32)
    for i in range(3):
        hrow = hash_ref[i:i + 1, :]
        srow = signs_ref[i:i + 1, :]
        e_t = jnp.where(m_i == hrow, 1.0, 0.0)
        s_i = lax.dot_general(wc * srow, e_t, (((1,), (1,)), ((), ())),
                              preferred_element_type=jnp.float32)
        res = res * s_i
    h3pre = lax.dot_general(res, w3_ref[...], (((1,), (1,)), ((), ())),
                            preferred_element_type=jnp.float32)
    yb = (h3pre + b3_ref[...]) * (g3_ref[...] * _BN_SCALE) + be3_ref[...]
    h3_ref[...] = _gelu_exact(yb)


def _stage2_body(vec_ref, w2_ref, b2_ref, g2_ref, be2_ref, out_ref):
    k = pl.program_id(1)
    col = k * KB + lax.broadcasted_iota(jnp.int32, (1, KB), 1)
    valid = col < SEC
    vb = jnp.where(valid, vec_ref[...], 0.0)
    wb = jnp.where(valid, w2_ref[...], 0.0)
    part = lax.dot_general(vb, wb, (((1,), (1,)), ((), ())),
                           preferred_element_type=jnp.float32)

    @pl.when(k == 0)
    def _():
        out_ref[...] = jnp.zeros_like(out_ref)

    out_ref[...] += part

    @pl.when(k == KSTEPS - 1)
    def _():
        acc = out_ref[...]
        yb = (acc + b2_ref[...]) * (g2_ref[...] * _BN_SCALE) + be2_ref[...]
        out_ref[...] = _gelu_exact(yb)


_TRIU_FLAT = None


def _triu_flat_idx():
    global _TRIU_FLAT
    if _TRIU_FLAT is None:
        iu0, iu1 = np.triu_indices(D)
        _TRIU_FLAT = np.asarray(iu0 * D + iu1, np.int32)
    return _TRIU_FLAT


def kernel(tokens, graph, signs, W2, b2, g2, be2, W3, b3, g3, be3, hash_idx):
    m2n, wc = pl.pallas_call(
        _stage1_body,
        grid=(B,),
        in_specs=[
            pl.BlockSpec((1, N, D), lambda b: (b, 0, 0)),
            pl.BlockSpec((1, N, N), lambda b: (b, 0, 0)),
        ],
        out_specs=[
            pl.BlockSpec((1, D, D), lambda b: (b, 0, 0)),
            pl.BlockSpec((1, 1, D), lambda b: (b, 0, 0)),
        ],
        out_shape=[
            jax.ShapeDtypeStruct((B, D, D), jnp.float32),
            jax.ShapeDtypeStruct((B, 1, D), jnp.float32),
        ],
        compiler_params=pltpu.CompilerParams(
            dimension_semantics=("parallel",),
        ),
        name="moment_stage1",
    )(tokens, graph)

    h3 = pl.pallas_call(
        _stage3_body,
        grid=(1,),
        in_specs=[
            pl.BlockSpec((B, D), lambda i: (0, 0)),
            pl.BlockSpec((3, D), lambda i: (0, 0)),
            pl.BlockSpec((3, D), lambda i: (0, 0)),
            pl.BlockSpec((DSEC, ESK), lambda i: (0, 0)),
            pl.BlockSpec((1, DSEC), lambda i: (0, 0)),
            pl.BlockSpec((1, DSEC), lambda i: (0, 0)),
            pl.BlockSpec((1, DSEC), lambda i: (0, 0)),
        ],
        out_specs=pl.BlockSpec((B, DSEC), lambda i: (0, 0)),
        out_shape=jax.ShapeDtypeStruct((B, DSEC), jnp.float32),
        compiler_params=pltpu.CompilerParams(
            dimension_semantics=("arbitrary",),
        ),
        name="moment_stage3",
    )(wc.reshape(B, D), signs.astype(jnp.float32),
      hash_idx.astype(jnp.int32), W3, b3.reshape(1, DSEC),
      g3.reshape(1, DSEC), be3.reshape(1, DSEC))

    vec = m2n.reshape(B, D * D)[:, _triu_flat_idx()]

    h2 = pl.pallas_call(
        _stage2_body,
        grid=(2, KSTEPS),
        in_specs=[
            pl.BlockSpec((B, KB), lambda h, k: (0, k)),
            pl.BlockSpec((DSEC // 2, KB), lambda h, k: (h, k)),
            pl.BlockSpec((1, DSEC // 2), lambda h, k: (0, h)),
            pl.BlockSpec((1, DSEC // 2), lambda h, k: (0, h)),
            pl.BlockSpec((1, DSEC // 2), lambda h, k: (0, h)),
        ],
        out_specs=pl.BlockSpec((B, DSEC // 2), lambda h, k: (0, h)),
        out_shape=jax.ShapeDtypeStruct((B, DSEC), jnp.float32),
        compiler_params=pltpu.CompilerParams(
            dimension_semantics=("parallel", "arbitrary"),
        ),
        name="moment_stage2",
    )(vec, W2, b2.reshape(1, DSEC), g2.reshape(1, DSEC),
      be2.reshape(1, DSEC))

    return jnp.concatenate([h2, h3], axis=-1)


# speedup vs baseline: 3.3639x; 1.1722x over previous
import numpy as np
import jax
import jax.numpy as jnp
from jax import lax
from jax.experimental import pallas as pl
from jax.experimental.pallas import tpu as pltpu

EPS = 1e-5
B, N, D = 64, 196, 384
ESK = 768
DSEC = 512
SEC = D * (D + 1) // 2
KB = 7424
KSTEPS = 10
_BN_SCALE = 1.0 / np.sqrt(1.0 + EPS)
_INV_SQRT2 = 1.0 / np.sqrt(2.0)


def _gelu_exact(y):
    return 0.5 * y * (1.0 + lax.erf(y * _INV_SQRT2))


VECP = 74304


def _stage1_body(tokens_ref, graph_ref, vecp_ref, wc_ref, y_scr):
    g = graph_ref[0]
    t = tokens_ref[0]
    deg = jnp.maximum(jnp.sum(g, axis=1, keepdims=True), EPS)
    dinv = lax.rsqrt(deg)
    gd = jnp.dot(g, dinv, preferred_element_type=jnp.float32)
    row = dinv * gd
    r_i = lax.broadcasted_iota(jnp.int32, (N, N), 0)
    c_i = lax.broadcasted_iota(jnp.int32, (N, N), 1)
    diag = jnp.sum(jnp.where(r_i == c_i, g, 0.0), axis=1, keepdims=True)
    trw = jnp.sum(diag * dinv * dinv)
    inv_trw = 1.0 / (trw + EPS)
    mu = lax.dot_general(row, t, (((0,), (0,)), ((), ())),
                         preferred_element_type=jnp.float32) * inv_trw
    tc = t - mu
    u = tc * dinv
    gu = jnp.dot(g, u, preferred_element_type=jnp.float32)
    m2 = lax.dot_general(u, gu, (((0,), (0,)), ((), ())),
                         preferred_element_type=jnp.float32)
    tr = jnp.sum(u * gu)
    z = m2 * (1.0 / (tr + EPS))
    rI = lax.broadcasted_iota(jnp.int32, (D, D), 0)
    cI = lax.broadcasted_iota(jnp.int32, (D, D), 1)
    eye = jnp.where(rI == cI, 1.0, 0.0)
    y = 1.5 * eye - 0.5 * z
    for _ in range(2):
        zy = jnp.dot(z, y, preferred_element_type=jnp.float32)
        y = 0.5 * jnp.dot(y, 3.0 * eye - zy,
                          preferred_element_type=jnp.float32)
    y_scr[...] = y * (1.0 / jnp.sqrt(tr + EPS))
    for i in range(D):
        off = i * D - (i * (i - 1)) // 2
        vecp_ref[0, 0:1, off:off + (D - i)] = y_scr[i:i + 1, i:D]

    wc = lax.dot_general(row, tc, (((0,), (0,)), ((), ())),
                         preferred_element_type=jnp.float32) * inv_trw
    wc_ref[0] = wc


def _stage3_body(wc_ref, signs_ref, hash_ref, w3_ref, b3_ref, g3_ref,
                 be3_ref, h3_ref):
    wc = wc_ref[...]
    m_i = lax.broadcasted_iota(jnp.int32, (ESK, D), 0)
    res = jnp.ones((B, ESK), jnp.float32)
    for i in range(3):
        hrow = hash_ref[i:i + 1, :]
        srow = signs_ref[i:i + 1, :]
        e_t = jnp.where(m_i == hrow, 1.0, 0.0)
        s_i = lax.dot_general(wc * srow, e_t, (((1,), (1,)), ((), ())),
                              preferred_element_type=jnp.float32)
        res = res * s_i
    h3pre = lax.dot_general(res, w3_ref[...], (((1,), (1,)), ((), ())),
                            preferred_element_type=jnp.float32)
    yb = (h3pre + b3_ref[...]) * (g3_ref[...] * _BN_SCALE) + be3_ref[...]
    h3_ref[...] = _gelu_exact(yb)


def _stage2_body(vec_ref, w2_ref, b2_ref, g2_ref, be2_ref, out_ref):
    k = pl.program_id(1)
    col = k * KB + lax.broadcasted_iota(jnp.int32, (1, KB), 1)
    valid = col < SEC
    vb = jnp.where(valid, vec_ref[...], 0.0)
    wb = jnp.where(valid, w2_ref[...], 0.0)
    part = lax.dot_general(vb, wb, (((1,), (1,)), ((), ())),
                           preferred_element_type=jnp.float32)

    @pl.when(k == 0)
    def _():
        out_ref[...] = jnp.zeros_like(out_ref)

    out_ref[...] += part

    @pl.when(k == KSTEPS - 1)
    def _():
        acc = out_ref[...]
        yb = (acc + b2_ref[...]) * (g2_ref[...] * _BN_SCALE) + be2_ref[...]
        out_ref[...] = _gelu_exact(yb)


def kernel(tokens, graph, signs, W2, b2, g2, be2, W3, b3, g3, be3, hash_idx):
    vecp, wc = pl.pallas_call(
        _stage1_body,
        grid=(B,),
        in_specs=[
            pl.BlockSpec((1, N, D), lambda b: (b, 0, 0)),
            pl.BlockSpec((1, N, N), lambda b: (b, 0, 0)),
        ],
        out_specs=[
            pl.BlockSpec((1, 1, VECP), lambda b: (b, 0, 0)),
            pl.BlockSpec((1, 1, D), lambda b: (b, 0, 0)),
        ],
        out_shape=[
            jax.ShapeDtypeStruct((B, 1, VECP), jnp.float32),
            jax.ShapeDtypeStruct((B, 1, D), jnp.float32),
        ],
        scratch_shapes=[pltpu.VMEM((D, D), jnp.float32)],
        compiler_params=pltpu.CompilerParams(
            dimension_semantics=("parallel",),
        ),
        name="moment_stage1",
    )(tokens, graph)

    h3 = pl.pallas_call(
        _stage3_body,
        grid=(1,),
        in_specs=[
            pl.BlockSpec((B, D), lambda i: (0, 0)),
            pl.BlockSpec((3, D), lambda i: (0, 0)),
            pl.BlockSpec((3, D), lambda i: (0, 0)),
            pl.BlockSpec((DSEC, ESK), lambda i: (0, 0)),
            pl.BlockSpec((1, DSEC), lambda i: (0, 0)),
            pl.BlockSpec((1, DSEC), lambda i: (0, 0)),
            pl.BlockSpec((1, DSEC), lambda i: (0, 0)),
        ],
        out_specs=pl.BlockSpec((B, DSEC), lambda i: (0, 0)),
        out_shape=jax.ShapeDtypeStruct((B, DSEC), jnp.float32),
        compiler_params=pltpu.CompilerParams(
            dimension_semantics=("arbitrary",),
        ),
        name="moment_stage3",
    )(wc.reshape(B, D), signs.astype(jnp.float32),
      hash_idx.astype(jnp.int32), W3, b3.reshape(1, DSEC),
      g3.reshape(1, DSEC), be3.reshape(1, DSEC))

    h2 = pl.pallas_call(
        _stage2_body,
        grid=(2, KSTEPS),
        in_specs=[
            pl.BlockSpec((B, KB), lambda h, k: (0, k)),
            pl.BlockSpec((DSEC // 2, KB), lambda h, k: (h, k)),
            pl.BlockSpec((1, DSEC // 2), lambda h, k: (0, h)),
            pl.BlockSpec((1, DSEC // 2), lambda h, k: (0, h)),
            pl.BlockSpec((1, DSEC // 2), lambda h, k: (0, h)),
        ],
        out_specs=pl.BlockSpec((B, DSEC // 2), lambda h, k: (0, h)),
        out_shape=jax.ShapeDtypeStruct((B, DSEC), jnp.float32),
        compiler_params=pltpu.CompilerParams(
            dimension_semantics=("parallel", "arbitrary"),
        ),
        name="moment_stage2",
    )(vecp.reshape(B, VECP), W2, b2.reshape(1, DSEC), g2.reshape(1, DSEC),
      be2.reshape(1, DSEC))

    return jnp.concatenate([h2, h3], axis=-1)


# speedup vs baseline: 3.4800x; 1.0345x over previous
import numpy as np
import jax
import jax.numpy as jnp
from jax import lax
from jax.experimental import pallas as pl
from jax.experimental.pallas import tpu as pltpu

EPS = 1e-5
B, N, D = 64, 196, 384
ESK = 768
DSEC = 512
SEC = D * (D + 1) // 2
KB = 7424
KSTEPS = 10
_BN_SCALE = 1.0 / np.sqrt(1.0 + EPS)
_INV_SQRT2 = 1.0 / np.sqrt(2.0)


def _gelu_exact(y):
    return 0.5 * y * (1.0 + lax.erf(y * _INV_SQRT2))


VECP = 74304


def _stage1_body(tokens_ref, graph_ref, vecp_ref, wc_ref, y_scr):
    g = graph_ref[0]
    t = tokens_ref[0]
    deg = jnp.maximum(jnp.sum(g, axis=1, keepdims=True), EPS)
    dinv = lax.rsqrt(deg)
    gd = jnp.dot(g, dinv, preferred_element_type=jnp.float32)
    row = dinv * gd
    r_i = lax.broadcasted_iota(jnp.int32, (N, N), 0)
    c_i = lax.broadcasted_iota(jnp.int32, (N, N), 1)
    diag = jnp.sum(jnp.where(r_i == c_i, g, 0.0), axis=1, keepdims=True)
    trw = jnp.sum(diag * dinv * dinv)
    inv_trw = 1.0 / (trw + EPS)
    mu = lax.dot_general(row, t, (((0,), (0,)), ((), ())),
                         preferred_element_type=jnp.float32) * inv_trw
    tc = t - mu
    u = tc * dinv
    gu = jnp.dot(g, u, preferred_element_type=jnp.float32)
    m2 = lax.dot_general(u, gu, (((0,), (0,)), ((), ())),
                         preferred_element_type=jnp.float32)
    tr = jnp.sum(u * gu)
    z = m2 * (1.0 / (tr + EPS))
    rI = lax.broadcasted_iota(jnp.int32, (D, D), 0)
    cI = lax.broadcasted_iota(jnp.int32, (D, D), 1)
    eye = jnp.where(rI == cI, 1.0, 0.0)
    y = 1.5 * eye - 0.5 * z
    for _ in range(2):
        zy = jnp.dot(z, y, preferred_element_type=jnp.float32)
        y = 0.5 * jnp.dot(y, 3.0 * eye - zy,
                          preferred_element_type=jnp.float32)
    y_scr[...] = y * (1.0 / jnp.sqrt(tr + EPS))
    for i in range(D):
        off = i * D - (i * (i - 1)) // 2
        vecp_ref[0, 0:1, off:off + (D - i)] = y_scr[i:i + 1, i:D]

    wc = lax.dot_general(row, tc, (((0,), (0,)), ((), ())),
                         preferred_element_type=jnp.float32) * inv_trw
    wc_ref[0] = wc


def _stage3_body(wc_ref, signs_ref, hash_ref, w3_ref, b3_ref, g3_ref,
                 be3_ref, h3_ref):
    wc = wc_ref[:, 0, :]
    m_i = lax.broadcasted_iota(jnp.int32, (ESK, D), 0)
    res = jnp.ones((B, ESK), jnp.float32)
    for i in range(3):
        hrow = hash_ref[i:i + 1, :]
        srow = signs_ref[i:i + 1, :]
        e_t = jnp.where(m_i == hrow, 1.0, 0.0)
        s_i = lax.dot_general(wc * srow, e_t, (((1,), (1,)), ((), ())),
                              preferred_element_type=jnp.float32)
        res = res * s_i
    h3pre = lax.dot_general(res, w3_ref[...], (((1,), (1,)), ((), ())),
                            preferred_element_type=jnp.float32)
    yb = (h3pre + b3_ref[...]) * (g3_ref[...] * _BN_SCALE) + be3_ref[...]
    h3_ref[...] = _gelu_exact(yb)


def _stage2_body(vec_ref, w2_ref, b2_ref, g2_ref, be2_ref, out_ref):
    k = pl.program_id(1)
    col = k * KB + lax.broadcasted_iota(jnp.int32, (1, KB), 1)
    valid = col < SEC
    vb = jnp.where(valid, vec_ref[:, 0, :], 0.0)
    wb = jnp.where(valid, w2_ref[...], 0.0)
    part = lax.dot_general(vb, wb, (((1,), (1,)), ((), ())),
                           preferred_element_type=jnp.float32)

    @pl.when(k == 0)
    def _():
        out_ref[...] = jnp.zeros_like(out_ref)

    out_ref[...] += part

    @pl.when(k == KSTEPS - 1)
    def _():
        acc = out_ref[...]
        yb = (acc + b2_ref[...]) * (g2_ref[...] * _BN_SCALE) + be2_ref[...]
        out_ref[...] = _gelu_exact(yb)


def kernel(tokens, graph, signs, W2, b2, g2, be2, W3, b3, g3, be3, hash_idx):
    vecp, wc = pl.pallas_call(
        _stage1_body,
        grid=(B,),
        in_specs=[
            pl.BlockSpec((1, N, D), lambda b: (b, 0, 0)),
            pl.BlockSpec((1, N, N), lambda b: (b, 0, 0)),
        ],
        out_specs=[
            pl.BlockSpec((1, 1, VECP), lambda b: (b, 0, 0)),
            pl.BlockSpec((1, 1, D), lambda b: (b, 0, 0)),
        ],
        out_shape=[
            jax.ShapeDtypeStruct((B, 1, VECP), jnp.float32),
            jax.ShapeDtypeStruct((B, 1, D), jnp.float32),
        ],
        scratch_shapes=[pltpu.VMEM((D, D), jnp.float32)],
        compiler_params=pltpu.CompilerParams(
            dimension_semantics=("parallel",),
        ),
        name="moment_stage1",
    )(tokens, graph)

    h3 = pl.pallas_call(
        _stage3_body,
        grid=(1,),
        in_specs=[
            pl.BlockSpec((B, 1, D), lambda i: (0, 0, 0)),
            pl.BlockSpec((3, D), lambda i: (0, 0)),
            pl.BlockSpec((3, D), lambda i: (0, 0)),
            pl.BlockSpec((DSEC, ESK), lambda i: (0, 0)),
            pl.BlockSpec((1, DSEC), lambda i: (0, 0)),
            pl.BlockSpec((1, DSEC), lambda i: (0, 0)),
            pl.BlockSpec((1, DSEC), lambda i: (0, 0)),
        ],
        out_specs=pl.BlockSpec((B, DSEC), lambda i: (0, 0)),
        out_shape=jax.ShapeDtypeStruct((B, DSEC), jnp.float32),
        compiler_params=pltpu.CompilerParams(
            dimension_semantics=("arbitrary",),
        ),
        name="moment_stage3",
    )(wc, signs.astype(jnp.float32),
      hash_idx.astype(jnp.int32), W3, b3.reshape(1, DSEC),
      g3.reshape(1, DSEC), be3.reshape(1, DSEC))

    h2 = pl.pallas_call(
        _stage2_body,
        grid=(2, KSTEPS),
        in_specs=[
            pl.BlockSpec((B, 1, KB), lambda h, k: (0, 0, k)),
            pl.BlockSpec((DSEC // 2, KB), lambda h, k: (h, k)),
            pl.BlockSpec((1, DSEC // 2), lambda h, k: (0, h)),
            pl.BlockSpec((1, DSEC // 2), lambda h, k: (0, h)),
            pl.BlockSpec((1, DSEC // 2), lambda h, k: (0, h)),
        ],
        out_specs=pl.BlockSpec((B, DSEC // 2), lambda h, k: (0, h)),
        out_shape=jax.ShapeDtypeStruct((B, DSEC), jnp.float32),
        compiler_params=pltpu.CompilerParams(
            dimension_semantics=("parallel", "arbitrary"),
        ),
        name="moment_stage2",
    )(vecp, W2, b2.reshape(1, DSEC), g2.reshape(1, DSEC),
      be2.reshape(1, DSEC))

    return jnp.concatenate([h2, h3], axis=-1)


# speedup vs baseline: 4.9131x; 1.4118x over previous
import numpy as np
import jax
import jax.numpy as jnp
from jax import lax
from jax.experimental import pallas as pl
from jax.experimental.pallas import tpu as pltpu

EPS = 1e-5
B, N, D = 64, 196, 384
ESK = 768
DSEC = 512
SEC = D * (D + 1) // 2
KB = 7424
KSTEPS = 10
_BN_SCALE = 1.0 / np.sqrt(1.0 + EPS)
_INV_SQRT2 = 1.0 / np.sqrt(2.0)


def _gelu_exact(y):
    return 0.5 * y * (1.0 + lax.erf(y * _INV_SQRT2))


VECP = 74304


def _stage1_body(tokens_ref, graph_ref, vecp_ref, wc_ref, y_scr):
    g = graph_ref[:, 0, 0, :]
    t = tokens_ref[:, 0, 0, :]
    deg = jnp.maximum(jnp.sum(g, axis=1, keepdims=True), EPS)
    dinv = lax.rsqrt(deg)
    gd = jnp.dot(g, dinv, preferred_element_type=jnp.float32)
    row = dinv * gd
    r_i = lax.broadcasted_iota(jnp.int32, (N, N), 0)
    c_i = lax.broadcasted_iota(jnp.int32, (N, N), 1)
    diag = jnp.sum(jnp.where(r_i == c_i, g, 0.0), axis=1, keepdims=True)
    trw = jnp.sum(diag * dinv * dinv)
    inv_trw = 1.0 / (trw + EPS)
    mu = lax.dot_general(row, t, (((0,), (0,)), ((), ())),
                         preferred_element_type=jnp.float32) * inv_trw
    tc = t - mu
    u = tc * dinv
    gu = jnp.dot(g, u, preferred_element_type=jnp.float32)
    m2 = lax.dot_general(u, gu, (((0,), (0,)), ((), ())),
                         preferred_element_type=jnp.float32)
    tr = jnp.sum(u * gu)
    z = m2 * (1.0 / (tr + EPS))
    rI = lax.broadcasted_iota(jnp.int32, (D, D), 0)
    cI = lax.broadcasted_iota(jnp.int32, (D, D), 1)
    eye = jnp.where(rI == cI, 1.0, 0.0)
    y = 1.5 * eye - 0.5 * z
    for _ in range(2):
        zy = jnp.dot(z, y, preferred_element_type=jnp.float32)
        y = 0.5 * jnp.dot(y, 3.0 * eye - zy,
                          preferred_element_type=jnp.float32)
    y_scr[...] = y * (1.0 / jnp.sqrt(tr + EPS))
    for i in range(D):
        off = i * D - (i * (i - 1)) // 2
        vecp_ref[0, 0:1, off:off + (D - i)] = y_scr[i:i + 1, i:D]

    wc = lax.dot_general(row, tc, (((0,), (0,)), ((), ())),
                         preferred_element_type=jnp.float32) * inv_trw
    wc_ref[0] = wc


def _stage3_body(wc_ref, signs_ref, hash_ref, w3_ref, b3_ref, g3_ref,
                 be3_ref, h3_ref):
    wc = wc_ref[:, 0, :]
    m_i = lax.broadcasted_iota(jnp.int32, (ESK, D), 0)
    res = jnp.ones((B, ESK), jnp.float32)
    for i in range(3):
        hrow = hash_ref[i:i + 1, :]
        srow = signs_ref[i:i + 1, :]
        e_t = jnp.where(m_i == hrow, 1.0, 0.0)
        s_i = lax.dot_general(wc * srow, e_t, (((1,), (1,)), ((), ())),
                              preferred_element_type=jnp.float32)
        res = res * s_i
    h3pre = lax.dot_general(res, w3_ref[...], (((1,), (1,)), ((), ())),
                            preferred_element_type=jnp.float32)
    yb = (h3pre + b3_ref[...]) * (g3_ref[...] * _BN_SCALE) + be3_ref[...]
    h3_ref[...] = _gelu_exact(yb)


def _stage2_body(vec_ref, w2_ref, b2_ref, g2_ref, be2_ref, out_ref):
    k = pl.program_id(1)
    col = k * KB + lax.broadcasted_iota(jnp.int32, (1, KB), 1)
    valid = col < SEC
    vb = jnp.where(valid, vec_ref[:, 0, :], 0.0)
    rowv = k * KB + lax.broadcasted_iota(jnp.int32, (KB, DSEC // 2), 0)
    wb = jnp.where(rowv < SEC, w2_ref[...], 0.0)
    part = lax.dot_general(vb, wb, (((1,), (0,)), ((), ())),
                           preferred_element_type=jnp.float32)

    @pl.when(k == 0)
    def _():
        out_ref[...] = jnp.zeros_like(out_ref)

    out_ref[...] += part

    @pl.when(k == KSTEPS - 1)
    def _():
        acc = out_ref[...]
        yb = (acc + b2_ref[...]) * (g2_ref[...] * _BN_SCALE) + be2_ref[...]
        out_ref[...] = _gelu_exact(yb)


def kernel(tokens, graph, signs, W2, b2, g2, be2, W3, b3, g3, be3, hash_idx):
    vecp, wc = pl.pallas_call(
        _stage1_body,
        grid=(B,),
        in_specs=[
            pl.BlockSpec((N, 1, 1, D), lambda b: (0, b, 0, 0)),
            pl.BlockSpec((N, 1, 1, N), lambda b: (0, b, 0, 0)),
        ],
        out_specs=[
            pl.BlockSpec((1, 1, VECP), lambda b: (b, 0, 0)),
            pl.BlockSpec((1, 1, D), lambda b: (b, 0, 0)),
        ],
        out_shape=[
            jax.ShapeDtypeStruct((B, 1, VECP), jnp.float32),
            jax.ShapeDtypeStruct((B, 1, D), jnp.float32),
        ],
        scratch_shapes=[pltpu.VMEM((D, D), jnp.float32)],
        compiler_params=pltpu.CompilerParams(
            dimension_semantics=("parallel",),
        ),
        name="moment_stage1",
    )(jnp.transpose(tokens, (1, 0, 2)).reshape(N, B, 1, D),
      jnp.transpose(graph, (1, 0, 2)).reshape(N, B, 1, N))

    h3 = pl.pallas_call(
        _stage3_body,
        grid=(1,),
        in_specs=[
            pl.BlockSpec((B, 1, D), lambda i: (0, 0, 0)),
            pl.BlockSpec((3, D), lambda i: (0, 0)),
            pl.BlockSpec((3, D), lambda i: (0, 0)),
            pl.BlockSpec((DSEC, ESK), lambda i: (0, 0)),
            pl.BlockSpec((1, DSEC), lambda i: (0, 0)),
            pl.BlockSpec((1, DSEC), lambda i: (0, 0)),
            pl.BlockSpec((1, DSEC), lambda i: (0, 0)),
        ],
        out_specs=pl.BlockSpec((B, DSEC), lambda i: (0, 0)),
        out_shape=jax.ShapeDtypeStruct((B, DSEC), jnp.float32),
        compiler_params=pltpu.CompilerParams(
            dimension_semantics=("arbitrary",),
        ),
        name="moment_stage3",
    )(wc, signs.astype(jnp.float32),
      hash_idx.astype(jnp.int32), W3, b3.reshape(1, DSEC),
      g3.reshape(1, DSEC), be3.reshape(1, DSEC))

    h2 = pl.pallas_call(
        _stage2_body,
        grid=(2, KSTEPS),
        in_specs=[
            pl.BlockSpec((B, 1, KB), lambda h, k: (0, 0, k)),
            pl.BlockSpec((KB, DSEC // 2), lambda h, k: (k, h)),
            pl.BlockSpec((1, DSEC // 2), lambda h, k: (0, h)),
            pl.BlockSpec((1, DSEC // 2), lambda h, k: (0, h)),
            pl.BlockSpec((1, DSEC // 2), lambda h, k: (0, h)),
        ],
        out_specs=pl.BlockSpec((B, DSEC // 2), lambda h, k: (0, h)),
        out_shape=jax.ShapeDtypeStruct((B, DSEC), jnp.float32),
        compiler_params=pltpu.CompilerParams(
            dimension_semantics=("parallel", "arbitrary"),
        ),
        name="moment_stage2",
    )(vecp, W2.T, b2.reshape(1, DSEC), g2.reshape(1, DSEC),
      be2.reshape(1, DSEC))

    return jnp.concatenate([h2, h3], axis=-1)


# speedup vs baseline: 6.2785x; 1.2779x over previous
import numpy as np
import jax
import jax.numpy as jnp
from jax import lax
from jax.experimental import pallas as pl
from jax.experimental.pallas import tpu as pltpu

EPS = 1e-5
B, N, D = 64, 196, 384
ESK = 768
DSEC = 512
SEC = D * (D + 1) // 2
KB = 7424
KSTEPS = 10
_BN_SCALE = 1.0 / np.sqrt(1.0 + EPS)
_INV_SQRT2 = 1.0 / np.sqrt(2.0)


def _gelu_exact(y):
    return 0.5 * y * (1.0 + lax.erf(y * _INV_SQRT2))


VECP = 74304


G = 8


def _stage1_body(tokens_ref, graph_ref, vecp_ref, wc_ref, y_scr):
    r_i = lax.broadcasted_iota(jnp.int32, (N, N), 0)
    c_i = lax.broadcasted_iota(jnp.int32, (N, N), 1)
    rI = lax.broadcasted_iota(jnp.int32, (D, D), 0)
    cI = lax.broadcasted_iota(jnp.int32, (D, D), 1)
    eye = jnp.where(rI == cI, 1.0, 0.0)
    for g in range(G):
        gm = graph_ref[:, g, :]
        t = tokens_ref[:, g, :]
        deg = jnp.maximum(jnp.sum(gm, axis=1, keepdims=True), EPS)
        dinv = lax.rsqrt(deg)
        gd = jnp.dot(gm, dinv, preferred_element_type=jnp.float32)
        row = dinv * gd
        diag = jnp.sum(jnp.where(r_i == c_i, gm, 0.0), axis=1, keepdims=True)
        trw = jnp.sum(diag * dinv * dinv)
        inv_trw = 1.0 / (trw + EPS)
        mu = lax.dot_general(row, t, (((0,), (0,)), ((), ())),
                             preferred_element_type=jnp.float32) * inv_trw
        tc = t - mu
        u = tc * dinv
        gu = jnp.dot(gm, u, preferred_element_type=jnp.float32)
        m2 = lax.dot_general(u, gu, (((0,), (0,)), ((), ())),
                             preferred_element_type=jnp.float32)
        tr = jnp.sum(u * gu)
        z = m2 * (1.0 / (tr + EPS))
        y = 1.5 * eye - 0.5 * z
        for _ in range(2):
            zy = jnp.dot(z, y, preferred_element_type=jnp.float32)
            y = 0.5 * jnp.dot(y, 3.0 * eye - zy,
                              preferred_element_type=jnp.float32)
        y_scr[g] = y * (1.0 / jnp.sqrt(tr + EPS))
        wc = lax.dot_general(row, tc, (((0,), (0,)), ((), ())),
                             preferred_element_type=jnp.float32) * inv_trw
        wc_ref[0, g:g + 1, :] = wc
    for i in range(D):
        off = i * D - (i * (i - 1)) // 2
        vecp_ref[0, :, off:off + (D - i)] = y_scr[:, i, i:D]


def _stage3_body(wc_ref, signs_ref, hash_ref, w3_ref, b3_ref, g3_ref,
                 be3_ref, h3_ref):
    wc = wc_ref[...]
    m_i = lax.broadcasted_iota(jnp.int32, (ESK, D), 0)
    res = jnp.ones((B, ESK), jnp.float32)
    for i in range(3):
        hrow = hash_ref[i:i + 1, :]
        srow = signs_ref[i:i + 1, :]
        e_t = jnp.where(m_i == hrow, 1.0, 0.0)
        s_i = lax.dot_general(wc * srow, e_t, (((1,), (1,)), ((), ())),
                              preferred_element_type=jnp.float32)
        res = res * s_i
    h3pre = lax.dot_general(res, w3_ref[...], (((1,), (1,)), ((), ())),
                            preferred_element_type=jnp.float32)
    yb = (h3pre + b3_ref[...]) * (g3_ref[...] * _BN_SCALE) + be3_ref[...]
    h3_ref[...] = _gelu_exact(yb)


def _stage2_body(vec_ref, w2_ref, b2_ref, g2_ref, be2_ref, out_ref):
    k = pl.program_id(1)
    col = k * KB + lax.broadcasted_iota(jnp.int32, (1, KB), 1)
    valid = col < SEC
    vb = jnp.where(valid, vec_ref[...], 0.0)
    rowv = k * KB + lax.broadcasted_iota(jnp.int32, (KB, DSEC // 2), 0)
    wb = jnp.where(rowv < SEC, w2_ref[...], 0.0)
    part = lax.dot_general(vb, wb, (((1,), (0,)), ((), ())),
                           preferred_element_type=jnp.float32)

    @pl.when(k == 0)
    def _():
        out_ref[...] = jnp.zeros_like(out_ref)

    out_ref[...] += part

    @pl.when(k == KSTEPS - 1)
    def _():
        acc = out_ref[...]
        yb = (acc + b2_ref[...]) * (g2_ref[...] * _BN_SCALE) + be2_ref[...]
        out_ref[...] = _gelu_exact(yb)


def kernel(tokens, graph, signs, W2, b2, g2, be2, W3, b3, g3, be3, hash_idx):
    vecp, wc = pl.pallas_call(
        _stage1_body,
        grid=(B // G,),
        in_specs=[
            pl.BlockSpec((N, G, D), lambda s: (0, s, 0)),
            pl.BlockSpec((N, G, N), lambda s: (0, s, 0)),
        ],
        out_specs=[
            pl.BlockSpec((1, G, VECP), lambda s: (s, 0, 0)),
            pl.BlockSpec((1, G, D), lambda s: (s, 0, 0)),
        ],
        out_shape=[
            jax.ShapeDtypeStruct((B // G, G, VECP), jnp.float32),
            jax.ShapeDtypeStruct((B // G, G, D), jnp.float32),
        ],
        scratch_shapes=[pltpu.VMEM((G, D, D), jnp.float32)],
        compiler_params=pltpu.CompilerParams(
            dimension_semantics=("arbitrary",),
        ),
        name="moment_stage1",
    )(jnp.transpose(tokens, (1, 0, 2)), jnp.transpose(graph, (1, 0, 2)))

    h3 = pl.pallas_call(
        _stage3_body,
        grid=(1,),
        in_specs=[
            pl.BlockSpec((B, D), lambda i: (0, 0)),
            pl.BlockSpec((3, D), lambda i: (0, 0)),
            pl.BlockSpec((3, D), lambda i: (0, 0)),
            pl.BlockSpec((DSEC, ESK), lambda i: (0, 0)),
            pl.BlockSpec((1, DSEC), lambda i: (0, 0)),
            pl.BlockSpec((1, DSEC), lambda i: (0, 0)),
            pl.BlockSpec((1, DSEC), lambda i: (0, 0)),
        ],
        out_specs=pl.BlockSpec((B, DSEC), lambda i: (0, 0)),
        out_shape=jax.ShapeDtypeStruct((B, DSEC), jnp.float32),
        compiler_params=pltpu.CompilerParams(
            dimension_semantics=("arbitrary",),
        ),
        name="moment_stage3",
    )(wc.reshape(B, D), signs.astype(jnp.float32),
      hash_idx.astype(jnp.int32), W3, b3.reshape(1, DSEC),
      g3.reshape(1, DSEC), be3.reshape(1, DSEC))

    h2 = pl.pallas_call(
        _stage2_body,
        grid=(2, KSTEPS),
        in_specs=[
            pl.BlockSpec((B, KB), lambda h, k: (0, k)),
            pl.BlockSpec((KB, DSEC // 2), lambda h, k: (k, h)),
            pl.BlockSpec((1, DSEC // 2), lambda h, k: (0, h)),
            pl.BlockSpec((1, DSEC // 2), lambda h, k: (0, h)),
            pl.BlockSpec((1, DSEC // 2), lambda h, k: (0, h)),
        ],
        out_specs=pl.BlockSpec((B, DSEC // 2), lambda h, k: (0, h)),
        out_shape=jax.ShapeDtypeStruct((B, DSEC), jnp.float32),
        compiler_params=pltpu.CompilerParams(
            dimension_semantics=("arbitrary", "arbitrary"),
        ),
        name="moment_stage2",
    )(vecp.reshape(B, VECP), W2.T, b2.reshape(1, DSEC), g2.reshape(1, DSEC),
      be2.reshape(1, DSEC))

    return jnp.concatenate([h2, h3], axis=-1)


# speedup vs baseline: 6.3356x; 1.0091x over previous
import numpy as np
import jax
import jax.numpy as jnp
from jax import lax
from jax.experimental import pallas as pl
from jax.experimental.pallas import tpu as pltpu

EPS = 1e-5
B, N, D = 64, 196, 384
ESK = 768
DSEC = 512
SEC = D * (D + 1) // 2
KB = 7424
KSTEPS = 10
_BN_SCALE = 1.0 / np.sqrt(1.0 + EPS)
_INV_SQRT2 = 1.0 / np.sqrt(2.0)


def _gelu_exact(y):
    return 0.5 * y * (1.0 + lax.erf(y * _INV_SQRT2))


VECP = 74304


G = 8


def _stage1_body(tokens_ref, graph_ref, vecp_ref, wc_ref, y_scr):
    r_i = lax.broadcasted_iota(jnp.int32, (N, N), 0)
    c_i = lax.broadcasted_iota(jnp.int32, (N, N), 1)
    rI = lax.broadcasted_iota(jnp.int32, (D, D), 0)
    cI = lax.broadcasted_iota(jnp.int32, (D, D), 1)
    eye = jnp.where(rI == cI, 1.0, 0.0)
    bf16 = jnp.bfloat16
    for g in range(G):
        gm = graph_ref[:, g, :]
        t = tokens_ref[:, g, :]
        deg = jnp.maximum(jnp.sum(gm, axis=1, keepdims=True), EPS)
        dinv = lax.rsqrt(deg)
        gd = jnp.dot(gm, dinv, preferred_element_type=jnp.float32)
        row = dinv * gd
        diag = jnp.sum(jnp.where(r_i == c_i, gm, 0.0), axis=1, keepdims=True)
        trw = jnp.sum(diag * dinv * dinv, axis=(0, 1), keepdims=True)
        inv_trw = 1.0 / (trw + EPS)
        mu = lax.dot_general(row, t, (((0,), (0,)), ((), ())),
                             preferred_element_type=jnp.float32) * inv_trw
        tc = t - mu
        wc = lax.dot_general(row, tc, (((0,), (0,)), ((), ())),
                             preferred_element_type=jnp.float32) * inv_trw
        wc_ref[0, g:g + 1, :] = wc
        u = tc * dinv
        gu = lax.dot_general(gm, u, (((1,), (0,)), ((), ())),
                             preferred_element_type=jnp.float32)
        m2 = lax.dot_general(u, gu, (((0,), (0,)), ((), ())),
                             preferred_element_type=jnp.float32)
        tr = jnp.sum(u * gu, axis=(0, 1), keepdims=True)
        z = m2 * (1.0 / (tr + EPS))
        y = 1.5 * eye - 0.5 * z
        for _ in range(2):
            zy = lax.dot_general(z, y, (((1,), (0,)), ((), ())),
                                 preferred_element_type=jnp.float32)
            y = 0.5 * lax.dot_general(y, 3.0 * eye - zy,
                                      (((1,), (0,)), ((), ())),
                                      preferred_element_type=jnp.float32)
        y_scr[g] = y * (1.0 / jnp.sqrt(tr + EPS))
    for i in range(D):
        off = i * D - (i * (i - 1)) // 2
        vecp_ref[0, :, off:off + (D - i)] = y_scr[:, i, i:D]


def _stage3_body(wc_ref, signs_ref, hash_ref, w3_ref, b3_ref, g3_ref,
                 be3_ref, h3_ref):
    wc = wc_ref[...]
    m_i = lax.broadcasted_iota(jnp.int32, (ESK, D), 0)
    res = jnp.ones((B, ESK), jnp.float32)
    for i in range(3):
        hrow = hash_ref[i:i + 1, :]
        srow = signs_ref[i:i + 1, :]
        e_t = jnp.where(m_i == hrow, 1.0, 0.0)
        s_i = lax.dot_general(wc * srow, e_t, (((1,), (1,)), ((), ())),
                              preferred_element_type=jnp.float32)
        res = res * s_i
    h3pre = lax.dot_general(res, w3_ref[...], (((1,), (1,)), ((), ())),
                            preferred_element_type=jnp.float32)
    yb = (h3pre + b3_ref[...]) * (g3_ref[...] * _BN_SCALE) + be3_ref[...]
    h3_ref[...] = _gelu_exact(yb)


def _stage2_body(vec_ref, w2_ref, b2_ref, g2_ref, be2_ref, out_ref):
    k = pl.program_id(1)
    col = k * KB + lax.broadcasted_iota(jnp.int32, (1, KB), 1)
    valid = col < SEC
    vb = jnp.where(valid, vec_ref[...], 0.0)
    rowv = k * KB + lax.broadcasted_iota(jnp.int32, (KB, DSEC // 2), 0)
    wb = jnp.where(rowv < SEC, w2_ref[...], 0.0)
    part = lax.dot_general(vb, wb, (((1,), (0,)), ((), ())),
                           preferred_element_type=jnp.float32)

    @pl.when(k == 0)
    def _():
        out_ref[...] = jnp.zeros_like(out_ref)

    out_ref[...] += part

    @pl.when(k == KSTEPS - 1)
    def _():
        acc = out_ref[...]
        yb = (acc + b2_ref[...]) * (g2_ref[...] * _BN_SCALE) + be2_ref[...]
        out_ref[...] = _gelu_exact(yb)


def kernel(tokens, graph, signs, W2, b2, g2, be2, W3, b3, g3, be3, hash_idx):
    vecp, wc = pl.pallas_call(
        _stage1_body,
        grid=(B // G,),
        in_specs=[
            pl.BlockSpec((N, G, D), lambda s: (0, s, 0)),
            pl.BlockSpec((N, G, N), lambda s: (0, s, 0)),
        ],
        out_specs=[
            pl.BlockSpec((1, G, VECP), lambda s: (s, 0, 0)),
            pl.BlockSpec((1, G, D), lambda s: (s, 0, 0)),
        ],
        out_shape=[
            jax.ShapeDtypeStruct((B // G, G, VECP), jnp.float32),
            jax.ShapeDtypeStruct((B // G, G, D), jnp.float32),
        ],
        scratch_shapes=[pltpu.VMEM((G, D, D), jnp.float32)],
        compiler_params=pltpu.CompilerParams(
            dimension_semantics=("arbitrary",),
        ),
        name="moment_stage1",
    )(jnp.transpose(tokens, (1, 0, 2)), jnp.transpose(graph, (1, 0, 2)))

    h3 = pl.pallas_call(
        _stage3_body,
        grid=(1,),
        in_specs=[
            pl.BlockSpec((B, D), lambda i: (0, 0)),
            pl.BlockSpec((3, D), lambda i: (0, 0)),
            pl.BlockSpec((3, D), lambda i: (0, 0)),
            pl.BlockSpec((DSEC, ESK), lambda i: (0, 0)),
            pl.BlockSpec((1, DSEC), lambda i: (0, 0)),
            pl.BlockSpec((1, DSEC), lambda i: (0, 0)),
            pl.BlockSpec((1, DSEC), lambda i: (0, 0)),
        ],
        out_specs=pl.BlockSpec((B, DSEC), lambda i: (0, 0)),
        out_shape=jax.ShapeDtypeStruct((B, DSEC), jnp.float32),
        compiler_params=pltpu.CompilerParams(
            dimension_semantics=("arbitrary",),
        ),
        name="moment_stage3",
    )(wc.reshape(B, D), signs.astype(jnp.float32),
      hash_idx.astype(jnp.int32), W3, b3.reshape(1, DSEC),
      g3.reshape(1, DSEC), be3.reshape(1, DSEC))

    h2 = pl.pallas_call(
        _stage2_body,
        grid=(2, KSTEPS),
        in_specs=[
            pl.BlockSpec((B, KB), lambda h, k: (0, k)),
            pl.BlockSpec((KB, DSEC // 2), lambda h, k: (k, h)),
            pl.BlockSpec((1, DSEC // 2), lambda h, k: (0, h)),
            pl.BlockSpec((1, DSEC // 2), lambda h, k: (0, h)),
            pl.BlockSpec((1, DSEC // 2), lambda h, k: (0, h)),
        ],
        out_specs=pl.BlockSpec((B, DSEC // 2), lambda h, k: (0, h)),
        out_shape=jax.ShapeDtypeStruct((B, DSEC), jnp.float32),
        compiler_params=pltpu.CompilerParams(
            dimension_semantics=("arbitrary", "arbitrary"),
        ),
        name="moment_stage2",
    )(vecp.reshape(B, VECP), W2.T, b2.reshape(1, DSEC), g2.reshape(1, DSEC),
      be2.reshape(1, DSEC))

    return jnp.concatenate([h2, h3], axis=-1)
